# baseline (speedup 1.0000x reference)
import jax
import jax.numpy as jnp
from jax import lax
from jax.experimental import pallas as pl
from jax.experimental.pallas import tpu as pltpu

_S = 30.0
_M = 0.35
_LANES = 128
_NQ = 4


def _stripe_body(n_rows, n_cols, cpq, *refs):
    xq = refs[:_NQ]
    gb_ref, laneq_ref, out_ref = refs[_NQ:]
    i = pl.program_id(0)
    nch = pl.cdiv(n_cols, _LANES)
    rem = n_cols - (nch - 1) * _LANES

    @pl.when(i == 0)
    def _init():
        out_ref[...] = jnp.zeros_like(out_ref)

    def chunk(gc, mask):
        q, c = divmod(gc, cpq)
        xc = xq[q][:, c * _LANES:(c + 1) * _LANES]
        if mask and gc == nch - 1 and rem != _LANES:
            lane = lax.broadcasted_iota(jnp.int32, xc.shape, 1)
            xc = jnp.where(lane < rem, xc, -jnp.inf)
        return xc

    gb = gb_ref[...]
    bm = chunk(0, True)
    tg = jnp.where(gb == 0, chunk(0, False), bm)
    for gc in range(1, nch):
        bm = jnp.maximum(bm, chunk(gc, True))
        tg = jnp.where(gb == gc, chunk(gc, False), tg)

    acc = jnp.exp(_S * (chunk(0, True) - bm))
    for gc in range(1, nch):
        acc = acc + jnp.exp(_S * (chunk(gc, True) - bm))

    mrow = jnp.max(bm, axis=1, keepdims=True)
    srow = jnp.sum(acc * jnp.exp(_S * (bm - mrow)), axis=1, keepdims=True)
    t = jnp.sum(jnp.where(laneq_ref[...] != 0.0, tg, 0.0),
                axis=1, keepdims=True)
    e1 = jnp.exp(_S * (t - mrow))
    e2 = jnp.exp(_S * (t - _M - mrow))
    s_corr = jnp.maximum(srow - e1, 0.0) + e2
    nll = jnp.log(s_corr) + _S * mrow - _S * (t - _M)
    out_ref[...] = out_ref[...] + \
        jnp.sum(nll, axis=(0, 1), keepdims=True) / n_rows


@jax.jit
def kernel(input, label):
    n_rows, n_cols = input.shape
    lbl = label.astype(jnp.int32)

    gb = jnp.broadcast_to((lbl // _LANES)[:, None], (n_rows, _LANES))
    laneq = (lbl[:, None] % _LANES ==
             jnp.arange(_LANES, dtype=jnp.int32)[None, :]).astype(jnp.float32)

    rb = 16
    nch = pl.cdiv(n_cols, _LANES)
    cpq = pl.cdiv(nch, _NQ)
    bq = cpq * _LANES
    body = lambda *refs: _stripe_body(n_rows, n_cols, cpq, *refs)
    x_specs = [
        pl.BlockSpec((rb, bq), lambda i, q=q: (i, q)) for q in range(_NQ)
    ]
    out = pl.pallas_call(
        body,
        grid=(n_rows // rb,),
        in_specs=x_specs + [
            pl.BlockSpec((rb, _LANES), lambda i: (i, 0)),
            pl.BlockSpec((rb, _LANES), lambda i: (i, 0)),
        ],
        out_specs=pl.BlockSpec((1, 1), lambda i: (0, 0)),
        out_shape=jax.ShapeDtypeStruct((1, 1), jnp.float32),
    )(*([input] * _NQ), gb, laneq)
    return out[0, 0]


# speedup vs baseline: 3.2023x; 3.2023x over previous
import jax
import jax.numpy as jnp
from jax import lax
from jax.experimental import pallas as pl
from jax.experimental.pallas import tpu as pltpu

_S = 30.0
_M = 0.35
_SUB = 8


def _stripe_body(n_rows, n_cls, n_blocks, bs, xt_ref, lblg_ref, lblm_ref,
                 out_ref, m_ref, s_ref, tg_ref):
    i = pl.program_id(0)
    ns = bs // _SUB
    ns_tail = (n_cls - (n_blocks - 1) * bs) // _SUB

    @pl.when(i == 0)
    def _init():
        m_ref[...] = jnp.full_like(m_ref, -jnp.inf)
        s_ref[...] = jnp.zeros_like(s_ref)
        tg_ref[...] = jnp.zeros_like(tg_ref)

    lblg = lblg_ref[...]

    def update(n_slices):
        m_old = m_ref[...]
        bm = m_old
        tg = tg_ref[...]
        for k in range(n_slices):
            ch = xt_ref[k * _SUB:(k + 1) * _SUB, :]
            bm = jnp.maximum(bm, ch)
            tg = jnp.where(lblg == i * ns + k, ch, tg)
        tg_ref[...] = tg
        acc = s_ref[...] * jnp.exp(_S * (m_old - bm))
        for k in range(n_slices):
            ch = xt_ref[k * _SUB:(k + 1) * _SUB, :]
            acc = acc + jnp.exp(_S * (ch - bm))
        s_ref[...] = acc
        m_ref[...] = bm

    @pl.when(i < n_blocks - 1)
    def _main():
        update(ns)

    @pl.when(i == n_blocks - 1)
    def _tail():
        update(ns_tail)

        m8 = m_ref[...]
        mrow = jnp.max(m8, axis=0, keepdims=True)
        srow = jnp.sum(s_ref[...] * jnp.exp(_S * (m8 - mrow)),
                       axis=0, keepdims=True)
        sub = lax.broadcasted_iota(jnp.int32, m8.shape, 0)
        t = jnp.sum(jnp.where(sub == lblm_ref[...], tg_ref[...], 0.0),
                    axis=0, keepdims=True)
        e1 = jnp.exp(_S * (t - mrow))
        e2 = jnp.exp(_S * (t - _M - mrow))
        s_corr = jnp.maximum(srow - e1, 0.0) + e2
        nll = jnp.log(s_corr) + _S * mrow - _S * (t - _M)
        out_ref[...] = jnp.sum(nll, axis=(0, 1), keepdims=True) / n_rows


@jax.jit
def kernel(input, label):
    n_rows, n_cls = input.shape
    xt = input.T
    lbl = label.astype(jnp.int32)
    lblg = (lbl // _SUB).reshape(1, n_rows)
    lblm = (lbl % _SUB).reshape(1, n_rows)

    bs = 2048
    n_blocks = pl.cdiv(n_cls, bs)
    body = lambda *refs: _stripe_body(n_rows, n_cls, n_blocks, bs, *refs)
    out = pl.pallas_call(
        body,
        grid=(n_blocks,),
        in_specs=[
            pl.BlockSpec((bs, n_rows), lambda i: (i, 0)),
            pl.BlockSpec((1, n_rows), lambda i: (0, 0)),
            pl.BlockSpec((1, n_rows), lambda i: (0, 0)),
        ],
        out_specs=pl.BlockSpec((1, 1), lambda i: (0, 0)),
        out_shape=jax.ShapeDtypeStruct((1, 1), jnp.float32),
        scratch_shapes=[
            pltpu.VMEM((_SUB, n_rows), jnp.float32),
            pltpu.VMEM((_SUB, n_rows), jnp.float32),
            pltpu.VMEM((_SUB, n_rows), jnp.float32),
        ],
    )(xt, lblg, lblm)
    return out[0, 0]


# speedup vs baseline: 3.4401x; 1.0743x over previous
import jax
import jax.numpy as jnp
from jax import lax
from jax.experimental import pallas as pl
from jax.experimental.pallas import tpu as pltpu

_S = 30.0
_M = 0.35
_SUB = 8
_C1 = _S * 1.4426950408889634


def _stripe_body(n_rows, n_cls, n_blocks, bs, xt_ref, lblg_ref, lblm_ref,
                 out_ref, m_ref, s_ref, tg_ref):
    i = pl.program_id(0)
    ns = bs // _SUB
    ns_tail = (n_cls - (n_blocks - 1) * bs) // _SUB

    @pl.when(i == 0)
    def _init():
        m_ref[...] = jnp.full_like(m_ref, -jnp.inf)
        s_ref[...] = jnp.zeros_like(s_ref)
        tg_ref[...] = jnp.zeros_like(tg_ref)

    lblg = lblg_ref[...]

    def update(n_slices):
        m_old = m_ref[...]
        bm = m_old
        tg = tg_ref[...]
        for k in range(n_slices):
            ch = xt_ref[k * _SUB:(k + 1) * _SUB, :]
            bm = jnp.maximum(bm, ch)
            tg = jnp.where(lblg == i * ns + k, ch, tg)
        tg_ref[...] = tg
        acc = s_ref[...] * jnp.exp2(_C1 * (m_old - bm))
        for k in range(n_slices):
            ch = xt_ref[k * _SUB:(k + 1) * _SUB, :]
            acc = acc + jnp.exp2(_C1 * (ch - bm))
        s_ref[...] = acc
        m_ref[...] = bm

    @pl.when(i < n_blocks - 1)
    def _main():
        update(ns)

    @pl.when(i == n_blocks - 1)
    def _tail():
        update(ns_tail)

        m8 = m_ref[...]
        mrow = jnp.max(m8, axis=0, keepdims=True)
        srow = jnp.sum(s_ref[...] * jnp.exp2(_C1 * (m8 - mrow)),
                       axis=0, keepdims=True)
        sub = lax.broadcasted_iota(jnp.int32, m8.shape, 0)
        t = jnp.sum(jnp.where(sub == lblm_ref[...], tg_ref[...], 0.0),
                    axis=0, keepdims=True)
        e1 = jnp.exp(_S * (t - mrow))
        e2 = jnp.exp(_S * (t - _M - mrow))
        s_corr = jnp.maximum(srow - e1, 0.0) + e2
        nll = jnp.log(s_corr) + _S * mrow - _S * (t - _M)
        out_ref[...] = jnp.sum(nll, axis=(0, 1), keepdims=True) / n_rows


@jax.jit
def kernel(input, label):
    n_rows, n_cls = input.shape
    xt = input.T
    lbl = label.astype(jnp.int32)
    lblg = jnp.broadcast_to((lbl // _SUB)[None, :], (_SUB, n_rows))
    lblm = jnp.broadcast_to((lbl % _SUB)[None, :], (_SUB, n_rows))

    bs = 2048
    n_blocks = pl.cdiv(n_cls, bs)
    body = lambda *refs: _stripe_body(n_rows, n_cls, n_blocks, bs, *refs)
    out = pl.pallas_call(
        body,
        grid=(n_blocks,),
        in_specs=[
            pl.BlockSpec((bs, n_rows), lambda i: (i, 0)),
            pl.BlockSpec((_SUB, n_rows), lambda i: (0, 0)),
            pl.BlockSpec((_SUB, n_rows), lambda i: (0, 0)),
        ],
        out_specs=pl.BlockSpec((1, 1), lambda i: (0, 0)),
        out_shape=jax.ShapeDtypeStruct((1, 1), jnp.float32),
        scratch_shapes=[
            pltpu.VMEM((_SUB, n_rows), jnp.float32),
            pltpu.VMEM((_SUB, n_rows), jnp.float32),
            pltpu.VMEM((_SUB, n_rows), jnp.float32),
        ],
    )(xt, lblg, lblm)
    return out[0, 0]
